# Initial kernel scaffold; baseline (speedup 1.0000x reference)
import jax
import jax.numpy as jnp
from jax.experimental import pallas as pl


def kernel(boxes, scores):
    raise NotImplementedError("write your pallas kernel here")


# speedup vs baseline: 24.9126x; 24.9126x over previous
import jax
import jax.numpy as jnp
from jax.experimental import pallas as pl
from jax.experimental.pallas import tpu as pltpu

N = 20000
MAX_OUT = 300
IOU_THRESH = 0.5
LANES = 128
ROWS = 160
NPAD = ROWS * LANES
NEG = -1e30


def _nms_body(x1_ref, y1_ref, x2_ref, y2_ref, sc_ref,
              ob_ref, os_ref, oi_ref, live_ref):
    live_ref[...] = sc_ref[...]
    ob_ref[...] = jnp.zeros_like(ob_ref)
    os_ref[...] = jnp.zeros_like(os_ref)
    oi_ref[...] = jnp.full_like(oi_ref, -1)

    r_iota = jax.lax.broadcasted_iota(jnp.int32, (ROWS, LANES), 0)
    c_iota = jax.lax.broadcasted_iota(jnp.int32, (ROWS, LANES), 1)
    idx2d = r_iota * LANES + c_iota
    lane_iota = jax.lax.broadcasted_iota(jnp.int32, (1, LANES), 1)
    four_iota = jax.lax.broadcasted_iota(jnp.int32, (1, 4), 1)

    x1 = x1_ref[...]
    y1 = y1_ref[...]
    x2 = x2_ref[...]
    y2 = y2_ref[...]
    area = (x2 - x1) * (y2 - y1)

    def body(k, _):
        live = live_ref[...]
        m = jnp.max(live)
        valid = m > -0.5
        pidx = jnp.min(jnp.where(live == m, idx2d, NPAD))
        r = pidx // LANES
        c = pidx % LANES
        lmask = lane_iota == c

        def pick(ref):
            row = ref[pl.ds(r, 1), :]
            return jnp.sum(jnp.where(lmask, row, 0.0))

        bx1 = pick(x1_ref)
        by1 = pick(y1_ref)
        bx2 = pick(x2_ref)
        by2 = pick(y2_ref)
        barea = (bx2 - bx1) * (by2 - by1)

        iw = jnp.maximum(jnp.minimum(bx2, x2) - jnp.maximum(bx1, x1), 0.0)
        ih = jnp.maximum(jnp.minimum(by2, y2) - jnp.maximum(by1, y1), 0.0)
        inter = iw * ih
        iou = inter / (barea + area - inter + 1e-6)
        kill = (iou > IOU_THRESH) | (idx2d == pidx)
        live_ref[...] = jnp.where(kill, NEG, live)

        os_ref[pl.ds(k, 1), :] = jnp.where(valid, m, 0.0).reshape(1, 1)
        oi_ref[pl.ds(k, 1), :] = jnp.where(valid, pidx, -1).reshape(1, 1)
        brow = jnp.where(four_iota == 0, bx1,
               jnp.where(four_iota == 1, by1,
               jnp.where(four_iota == 2, bx2, by2)))
        ob_ref[pl.ds(k, 1), :] = jnp.where(valid, brow, 0.0)
        return 0

    jax.lax.fori_loop(0, MAX_OUT, body, 0)


def _run(x1, y1, x2, y2, sc, interpret=False):
    return pl.pallas_call(
        _nms_body,
        out_shape=(
            jax.ShapeDtypeStruct((MAX_OUT, 4), jnp.float32),
            jax.ShapeDtypeStruct((MAX_OUT, 1), jnp.float32),
            jax.ShapeDtypeStruct((MAX_OUT, 1), jnp.int32),
        ),
        scratch_shapes=[pltpu.VMEM((ROWS, LANES), jnp.float32)],
        interpret=interpret,
    )(x1, y1, x2, y2, sc)


@jax.jit
def kernel(boxes, scores):
    pad = NPAD - N
    x1 = jnp.pad(boxes[:, 0], (0, pad)).reshape(ROWS, LANES)
    y1 = jnp.pad(boxes[:, 1], (0, pad)).reshape(ROWS, LANES)
    x2 = jnp.pad(boxes[:, 2], (0, pad)).reshape(ROWS, LANES)
    y2 = jnp.pad(boxes[:, 3], (0, pad)).reshape(ROWS, LANES)
    sc = jnp.pad(scores, (0, pad), constant_values=NEG).reshape(ROWS, LANES)
    ob, os_, oi = _run(x1, y1, x2, y2, sc)
    return ob, os_.reshape(MAX_OUT), oi.reshape(MAX_OUT)


# speedup vs baseline: 30.2289x; 1.2134x over previous
import jax
import jax.numpy as jnp
from jax.experimental import pallas as pl
from jax.experimental.pallas import tpu as pltpu

N = 20000
MAX_OUT = 300
IOU_THRESH = 0.5
LANES = 128
ROWS = 160
NPAD = ROWS * LANES
NEG = -1e30
T = 4


def _nms_body(x1_ref, y1_ref, x2_ref, y2_ref, sc_ref,
              ob_ref, os_ref, oi_ref, live_ref, area_ref):
    live_ref[...] = sc_ref[...]
    area_ref[...] = (x2_ref[...] - x1_ref[...]) * (y2_ref[...] - y1_ref[...])
    ob_ref[...] = jnp.zeros_like(ob_ref)
    os_ref[...] = jnp.zeros_like(os_ref)
    oi_ref[...] = jnp.full_like(oi_ref, -1)

    r_iota = jax.lax.broadcasted_iota(jnp.int32, (ROWS, LANES), 0)
    c_iota = jax.lax.broadcasted_iota(jnp.int32, (ROWS, LANES), 1)
    idx2d = r_iota * LANES + c_iota
    lane_iota = jax.lax.broadcasted_iota(jnp.int32, (1, LANES), 1)
    four_iota = jax.lax.broadcasted_iota(jnp.int32, (1, 4), 1)

    def round_body(carry):
        k, mtop = carry
        excl = live_ref[...]

        ms, ps, bxs = [], [], []
        m_j = mtop
        for j in range(T):
            if j > 0:
                m_j = jnp.max(excl)
            p_j = jnp.min(jnp.where(excl == m_j, idx2d, NPAD))
            excl = jnp.where(idx2d == p_j, NEG, excl)
            r = p_j // LANES
            c = p_j % LANES
            lmask = lane_iota == c
            bx1 = jnp.sum(jnp.where(lmask, x1_ref[pl.ds(r, 1), :], 0.0))
            by1 = jnp.sum(jnp.where(lmask, y1_ref[pl.ds(r, 1), :], 0.0))
            bx2 = jnp.sum(jnp.where(lmask, x2_ref[pl.ds(r, 1), :], 0.0))
            by2 = jnp.sum(jnp.where(lmask, y2_ref[pl.ds(r, 1), :], 0.0))
            ms.append(m_j)
            ps.append(p_j)
            bxs.append((bx1, by1, bx2, by2, (bx2 - bx1) * (by2 - by1)))

        def pair_iou(i, j):
            ax1, ay1, ax2, ay2, aa = bxs[i]
            bx1, by1, bx2, by2, ba = bxs[j]
            iw = jnp.maximum(jnp.minimum(ax2, bx2) - jnp.maximum(ax1, bx1), 0.0)
            ih = jnp.maximum(jnp.minimum(ay2, by2) - jnp.maximum(ay1, by1), 0.0)
            inter = iw * ih
            return inter / (aa + ba - inter + 1e-6)

        committed = [ms[0] > -0.5]
        for j in range(1, T):
            ok = ms[j] > -0.5
            for i in range(j):
                ok = ok & (jnp.logical_not(committed[i])
                           | jnp.logical_not(pair_iou(i, j) > IOU_THRESH))
            committed.append(ok)

        x1 = x1_ref[...]
        y1 = y1_ref[...]
        x2 = x2_ref[...]
        y2 = y2_ref[...]
        area = area_ref[...]
        newlive = live_ref[...]
        for j in range(T):
            bx1, by1, bx2, by2, ba = bxs[j]
            iw = jnp.maximum(jnp.minimum(bx2, x2) - jnp.maximum(bx1, x1), 0.0)
            ih = jnp.maximum(jnp.minimum(by2, y2) - jnp.maximum(by1, y1), 0.0)
            inter = iw * ih
            iou = inter / (ba + area - inter + 1e-6)
            kill = ((iou > IOU_THRESH) & committed[j]) | (idx2d == ps[j])
            newlive = jnp.where(kill, NEG, newlive)
        live_ref[...] = newlive
        mnext = jnp.max(newlive)

        slot = k
        for j in range(T):
            bx1, by1, bx2, by2, ba = bxs[j]
            sclamp = jnp.minimum(slot, MAX_OUT - 1)

            @pl.when(committed[j] & (slot < MAX_OUT))
            def _():
                os_ref[pl.ds(sclamp, 1), :] = ms[j].reshape(1, 1)
                oi_ref[pl.ds(sclamp, 1), :] = ps[j].reshape(1, 1)
                brow = jnp.where(four_iota == 0, bx1,
                       jnp.where(four_iota == 1, by1,
                       jnp.where(four_iota == 2, bx2, by2)))
                ob_ref[pl.ds(sclamp, 1), :] = brow

            slot = slot + committed[j].astype(jnp.int32)

        return slot, mnext

    def cond(carry):
        k, mtop = carry
        return (k < MAX_OUT) & (mtop > -0.5)

    m0 = jnp.max(sc_ref[...])
    jax.lax.while_loop(cond, round_body, (jnp.int32(0), m0))


def _run(x1, y1, x2, y2, sc, interpret=False):
    return pl.pallas_call(
        _nms_body,
        out_shape=(
            jax.ShapeDtypeStruct((MAX_OUT, 4), jnp.float32),
            jax.ShapeDtypeStruct((MAX_OUT, 1), jnp.float32),
            jax.ShapeDtypeStruct((MAX_OUT, 1), jnp.int32),
        ),
        scratch_shapes=[pltpu.VMEM((ROWS, LANES), jnp.float32),
                        pltpu.VMEM((ROWS, LANES), jnp.float32)],
        interpret=interpret,
    )(x1, y1, x2, y2, sc)


@jax.jit
def kernel(boxes, scores):
    pad = NPAD - N
    x1 = jnp.pad(boxes[:, 0], (0, pad)).reshape(ROWS, LANES)
    y1 = jnp.pad(boxes[:, 1], (0, pad)).reshape(ROWS, LANES)
    x2 = jnp.pad(boxes[:, 2], (0, pad)).reshape(ROWS, LANES)
    y2 = jnp.pad(boxes[:, 3], (0, pad)).reshape(ROWS, LANES)
    sc = jnp.pad(scores, (0, pad), constant_values=NEG).reshape(ROWS, LANES)
    ob, os_, oi = _run(x1, y1, x2, y2, sc)
    return ob, os_.reshape(MAX_OUT), oi.reshape(MAX_OUT)


# speedup vs baseline: 34.7950x; 1.1510x over previous
import jax
import jax.numpy as jnp
from jax.experimental import pallas as pl
from jax.experimental.pallas import tpu as pltpu

N = 20000
MAX_OUT = 300
IOU_THRESH = 0.5
LANES = 128
ROWS = 160
NPAD = ROWS * LANES
NEG = -1e30
T = 4


def _tree(a, op2, red):
    a = op2(a[0:80], a[80:160])
    a = op2(a[0:40], a[40:80])
    a = red(a, axis=0, keepdims=True)
    return red(a, axis=1, keepdims=True)


def _tmax(a):
    return _tree(a, jnp.maximum, jnp.max)


def _tmin(a):
    return _tree(a, jnp.minimum, jnp.min)


def _tsum(a):
    return _tree(a, jnp.add, jnp.sum)


def _nms_body(x1_ref, y1_ref, x2_ref, y2_ref, sc_ref,
              ob_ref, os_ref, oi_ref, live_ref, area_ref):
    live_ref[...] = sc_ref[...]
    area_ref[...] = (x2_ref[...] - x1_ref[...]) * (y2_ref[...] - y1_ref[...])
    ob_ref[...] = jnp.zeros_like(ob_ref)
    os_ref[...] = jnp.zeros_like(os_ref)
    oi_ref[...] = jnp.full_like(oi_ref, -1)

    r_iota = jax.lax.broadcasted_iota(jnp.int32, (ROWS, LANES), 0)
    c_iota = jax.lax.broadcasted_iota(jnp.int32, (ROWS, LANES), 1)
    idx2d = r_iota * LANES + c_iota
    four_iota = jax.lax.broadcasted_iota(jnp.int32, (1, 4), 1)

    def round_body(carry):
        k, rnd = carry
        excl = live_ref[...]
        x1 = x1_ref[...]
        y1 = y1_ref[...]
        x2 = x2_ref[...]
        y2 = y2_ref[...]
        area = area_ref[...]

        ms, ps, bxs = [], [], []
        for j in range(T):
            m_j = _tmax(excl)
            p_j = _tmin(jnp.where(excl == m_j, idx2d, NPAD))
            eq_p = idx2d == p_j
            if j + 1 < T:
                excl = jnp.where(eq_p, NEG, excl)
            onehot = jnp.where(eq_p, 1.0, 0.0)
            bx1 = _tsum(onehot * x1)
            by1 = _tsum(onehot * y1)
            bx2 = _tsum(onehot * x2)
            by2 = _tsum(onehot * y2)
            ms.append(m_j)
            ps.append(p_j)
            bxs.append((bx1, by1, bx2, by2, (bx2 - bx1) * (by2 - by1)))

        def pair_iou(i, j):
            ax1, ay1, ax2, ay2, aa = bxs[i]
            bx1, by1, bx2, by2, ba = bxs[j]
            iw = jnp.maximum(jnp.minimum(ax2, bx2) - jnp.maximum(ax1, bx1), 0.0)
            ih = jnp.maximum(jnp.minimum(ay2, by2) - jnp.maximum(ay1, by1), 0.0)
            inter = iw * ih
            return inter / (aa + ba - inter + 1e-6)

        committed = [ms[0] > -0.5]
        for j in range(1, T):
            ok = ms[j] > -0.5
            for i in range(j):
                ok = ok & (jnp.logical_not(committed[i])
                           | jnp.logical_not(pair_iou(i, j) > IOU_THRESH))
            committed.append(ok)

        pk = committed[0].astype(jnp.int32)
        for j in range(1, T):
            pk = pk + committed[j].astype(jnp.int32) * (1 << j)
        pks = pk[0, 0]

        newlive = live_ref[...]
        for j in range(T):
            bx1, by1, bx2, by2, ba = bxs[j]
            iw = jnp.maximum(jnp.minimum(bx2, x2) - jnp.maximum(bx1, x1), 0.0)
            ih = jnp.maximum(jnp.minimum(by2, y2) - jnp.maximum(by1, y1), 0.0)
            inter = iw * ih
            iou = inter / (ba + area - inter + 1e-6)
            kill = ((iou > IOU_THRESH) & committed[j]) | (idx2d == ps[j])
            newlive = jnp.where(kill, NEG, newlive)
        live_ref[...] = newlive

        slot = k
        for j in range(T):
            bx1, by1, bx2, by2, ba = bxs[j]
            cbit = (pks >> j) & 1

            @pl.when((cbit == 1) & (slot < MAX_OUT))
            def _():
                sclamp = jnp.minimum(slot, MAX_OUT - 1)
                os_ref[pl.ds(sclamp, 1), :] = ms[j]
                oi_ref[pl.ds(sclamp, 1), :] = ps[j]
                brow = jnp.where(four_iota == 0, bxs[j][0],
                       jnp.where(four_iota == 1, bxs[j][1],
                       jnp.where(four_iota == 2, bxs[j][2], bxs[j][3])))
                ob_ref[pl.ds(sclamp, 1), :] = brow

            slot = slot + cbit

        return slot, rnd + 1

    def cond(carry):
        k, rnd = carry
        return (k < MAX_OUT) & (rnd < MAX_OUT)

    jax.lax.while_loop(cond, round_body, (jnp.int32(0), jnp.int32(0)))


def _run(x1, y1, x2, y2, sc, interpret=False):
    return pl.pallas_call(
        _nms_body,
        out_shape=(
            jax.ShapeDtypeStruct((MAX_OUT, 4), jnp.float32),
            jax.ShapeDtypeStruct((MAX_OUT, 1), jnp.float32),
            jax.ShapeDtypeStruct((MAX_OUT, 1), jnp.int32),
        ),
        scratch_shapes=[pltpu.VMEM((ROWS, LANES), jnp.float32),
                        pltpu.VMEM((ROWS, LANES), jnp.float32)],
        interpret=interpret,
    )(x1, y1, x2, y2, sc)


@jax.jit
def kernel(boxes, scores):
    pad = NPAD - N
    x1 = jnp.pad(boxes[:, 0], (0, pad)).reshape(ROWS, LANES)
    y1 = jnp.pad(boxes[:, 1], (0, pad)).reshape(ROWS, LANES)
    x2 = jnp.pad(boxes[:, 2], (0, pad)).reshape(ROWS, LANES)
    y2 = jnp.pad(boxes[:, 3], (0, pad)).reshape(ROWS, LANES)
    sc = jnp.pad(scores, (0, pad), constant_values=NEG).reshape(ROWS, LANES)
    ob, os_, oi = _run(x1, y1, x2, y2, sc)
    return ob, os_.reshape(MAX_OUT), oi.reshape(MAX_OUT)
